# Initial kernel scaffold; baseline (speedup 1.0000x reference)
import jax
import jax.numpy as jnp
from jax.experimental import pallas as pl


def kernel(x, edge_index, batch_size):
    raise NotImplementedError("write your pallas kernel here")


# speedup vs baseline: 4.2450x; 4.2450x over previous
import functools

import jax
import jax.numpy as jnp
from jax import lax
from jax.experimental import pallas as pl
from jax.experimental.pallas import tpu as pltpu
from jax.experimental.pallas import tpu_sc as plsc

N_NODES = 10000
D_FEAT = 128
N_EDGES = 320000
BATCH = 4096
H = D_FEAT // 2
NC, NS, L = 2, 16, 16
EPT = N_EDGES // NS
KB = 128
EPT_P = ((EPT + KB - 1) // KB) * KB
NB = EPT_P // KB
ACC_ROWS = BATCH + L
DUMP = BATCH
ZROWS = ACC_ROWS // NS
RPT = BATCH // NS


@functools.partial(
    pl.kernel,
    out_type=jax.ShapeDtypeStruct((4, BATCH, H), jnp.float32),
    mesh=plsc.VectorSubcoreMesh(core_axis_name="c", subcore_axis_name="s"),
    compiler_params=pltpu.CompilerParams(use_tc_tiling_on_sc=False),
    scratch_types=[
        pltpu.VMEM((KB,), jnp.int32),
        pltpu.VMEM((KB,), jnp.int32),
        pltpu.VMEM((KB, H), jnp.float32),
        pltpu.VMEM((RPT,), jnp.int32),
        pltpu.VMEM((RPT, H), jnp.float32),
        pltpu.VMEM((L,), jnp.int32),
        pltpu.VMEM_SHARED((ACC_ROWS, H), jnp.float32),
        pltpu.SemaphoreType.DMA,
    ],
)
def _agg_kernel(x2, src_p, dst_p, startv, zrows, out,
                bidx, bdst, rows, hidx, hrows, sv, acc, sem):
    c = lax.axis_index("c")
    s = lax.axis_index("s")

    pltpu.sync_copy(zrows, acc.at[pl.ds(s * ZROWS, ZROWS)])
    plsc.subcore_barrier()

    def batch_body(b, _):
        off = s * EPT_P + b * KB
        pltpu.sync_copy(src_p.at[pl.ds(off, KB)], bidx)
        pltpu.sync_copy(dst_p.at[pl.ds(off, KB)], bdst)
        for j in range(KB // L):
            sl = pl.ds(j * L, L)
            bidx[sl] = bidx[sl] * 2 + c
            dv = bdst[sl]
            inr = (dv >= 0) & (dv < BATCH)
            bdst[sl] = jnp.where(inr, dv, DUMP)
        pltpu.async_copy(x2.at[bidx], rows, sem).wait()
        pltpu.sync_copy(rows, acc.at[bdst], add=True)
        return 0

    lax.fori_loop(0, NB, batch_body, 0)

    plsc.subcore_barrier()

    pltpu.sync_copy(startv, sv)
    start = sv[...]
    iota = lax.iota(jnp.int32, L)
    base = s * RPT
    for j in range(RPT // L):
        hidx[pl.ds(j * L, L)] = (start + base + j * L + iota) * 2 + c
    pltpu.async_copy(x2.at[hidx], hrows, sem).wait()
    pltpu.sync_copy(hrows, out.at[c, pl.ds(base, RPT), :])
    pltpu.sync_copy(acc.at[pl.ds(base, RPT)],
                    out.at[2 + c, pl.ds(base, RPT), :])


def kernel(x, edge_index, batch_size):
    x = x.astype(jnp.float32)
    ei = edge_index.astype(jnp.int32)
    start = jnp.asarray(batch_size, jnp.int32) - BATCH
    src = ei[0]
    dstp = ei[1] - start
    src_p = jnp.pad(src.reshape(NS, EPT), ((0, 0), (0, EPT_P - EPT))).reshape(-1)
    dst_p = jnp.pad(dstp.reshape(NS, EPT), ((0, 0), (0, EPT_P - EPT)),
                    constant_values=-1).reshape(-1)
    x2 = x.reshape(2 * N_NODES, H)
    startv = jnp.full((L,), start, jnp.int32)
    zrows = jnp.zeros((ZROWS, H), jnp.float32)
    planes = _agg_kernel(x2, src_p, dst_p, startv, zrows)
    return planes.transpose(1, 0, 2).reshape(BATCH, 2 * D_FEAT)


# speedup vs baseline: 4.5372x; 1.0689x over previous
import functools

import jax
import jax.numpy as jnp
from jax import lax
from jax.experimental import pallas as pl
from jax.experimental.pallas import tpu as pltpu
from jax.experimental.pallas import tpu_sc as plsc

N_NODES = 10000
D_FEAT = 128
N_EDGES = 320000
BATCH = 4096
H = D_FEAT // 2
NC, NS, L = 2, 16, 16
EPT = N_EDGES // NS
KB = 128
NB = 158
EPT_P = NB * KB
ACC_ROWS = BATCH + L
DUMP = BATCH
ZROWS = ACC_ROWS // NS
RPT = BATCH // NS


@functools.partial(
    pl.kernel,
    out_type=jax.ShapeDtypeStruct((4, BATCH, H), jnp.float32),
    mesh=plsc.VectorSubcoreMesh(core_axis_name="c", subcore_axis_name="s"),
    compiler_params=pltpu.CompilerParams(use_tc_tiling_on_sc=False),
    scratch_types=[
        pltpu.VMEM((KB,), jnp.int32),
        pltpu.VMEM((KB,), jnp.int32),
        pltpu.VMEM((KB,), jnp.int32),
        pltpu.VMEM((KB,), jnp.int32),
        pltpu.VMEM((KB, H), jnp.float32),
        pltpu.VMEM((KB, H), jnp.float32),
        pltpu.VMEM((RPT,), jnp.int32),
        pltpu.VMEM((RPT, H), jnp.float32),
        pltpu.VMEM((L,), jnp.int32),
        pltpu.VMEM_SHARED((ACC_ROWS, H), jnp.float32),
        pltpu.SemaphoreType.DMA,
        pltpu.SemaphoreType.DMA,
        pltpu.SemaphoreType.DMA,
        pltpu.SemaphoreType.DMA,
        pltpu.SemaphoreType.DMA,
    ],
)
def _agg_kernel(x2, src_p, dst_p, startv, zrows, out,
                bidx0, bidx1, bdst0, bdst1, rows0, rows1,
                hidx, hrows, sv, acc,
                isem0, isem1, gsem, ssem0, ssem1):
    c = lax.axis_index("c")
    s = lax.axis_index("s")
    bidx = (bidx0, bidx1)
    bdst = (bdst0, bdst1)
    rows = (rows0, rows1)
    isem = (isem0, isem1)
    ssem = (ssem0, ssem1)

    pltpu.sync_copy(zrows, acc.at[pl.ds(s * ZROWS, ZROWS)])
    plsc.subcore_barrier()

    def start_idx(t, p):
        off = s * EPT_P + jnp.minimum(t, NB - 1) * KB
        pltpu.async_copy(src_p.at[pl.ds(off, KB)], bidx[p], isem[p])
        pltpu.async_copy(dst_p.at[pl.ds(off, KB)], bdst[p], isem[p])

    def wait_idx(p):
        pltpu.make_async_copy(src_p.at[pl.ds(0, KB)], bidx[p], isem[p]).wait()
        pltpu.make_async_copy(dst_p.at[pl.ds(0, KB)], bdst[p], isem[p]).wait()

    def remap(p):
        for j in range(KB // L):
            sl = pl.ds(j * L, L)
            bidx[p][sl] = bidx[p][sl] * 2 + c
            dv = bdst[p][sl]
            inr = (dv >= 0) & (dv < BATCH)
            bdst[p][sl] = jnp.where(inr, dv, DUMP)

    def step(t, p, first):
        wait_idx(p)
        remap(p)
        gd = pltpu.async_copy(x2.at[bidx[p]], rows[p], gsem)
        gd.wait()
        if not first:
            pltpu.make_async_copy(rows[1 - p], acc.at[bdst[1 - p]],
                                  ssem[1 - p]).wait()
        pltpu.async_copy(rows[p], acc.at[bdst[p]], ssem[p], add=True)
        start_idx(t + 1, 1 - p)

    start_idx(jnp.int32(0), 0)
    step(jnp.int32(0), 0, True)
    step(jnp.int32(1), 1, False)

    def pair_body(i, _):
        step(2 * i, 0, False)
        step(2 * i + 1, 1, False)
        return 0

    lax.fori_loop(1, NB // 2, pair_body, 0)

    pltpu.make_async_copy(rows[1], acc.at[bdst[1]], ssem[1]).wait()
    wait_idx(0)

    plsc.subcore_barrier()

    pltpu.sync_copy(startv, sv)
    start = sv[...]
    iota = lax.iota(jnp.int32, L)
    base = s * RPT
    for j in range(RPT // L):
        hidx[pl.ds(j * L, L)] = (start + base + j * L + iota) * 2 + c
    pltpu.async_copy(x2.at[hidx], hrows, gsem).wait()
    pltpu.sync_copy(hrows, out.at[c, pl.ds(base, RPT), :])
    pltpu.sync_copy(acc.at[pl.ds(base, RPT)],
                    out.at[2 + c, pl.ds(base, RPT), :])


def kernel(x, edge_index, batch_size):
    x = x.astype(jnp.float32)
    ei = edge_index.astype(jnp.int32)
    start = jnp.asarray(batch_size, jnp.int32) - BATCH
    src = ei[0]
    dstp = ei[1] - start
    src_p = jnp.pad(src.reshape(NS, EPT), ((0, 0), (0, EPT_P - EPT))).reshape(-1)
    dst_p = jnp.pad(dstp.reshape(NS, EPT), ((0, 0), (0, EPT_P - EPT)),
                    constant_values=-1).reshape(-1)
    x2 = x.reshape(2 * N_NODES, H)
    startv = jnp.full((L,), start, jnp.int32)
    zrows = jnp.zeros((ZROWS, H), jnp.float32)
    planes = _agg_kernel(x2, src_p, dst_p, startv, zrows)
    return planes.transpose(1, 0, 2).reshape(BATCH, 2 * D_FEAT)
